# Initial kernel scaffold; baseline (speedup 1.0000x reference)
import jax
import jax.numpy as jnp
from jax.experimental import pallas as pl


def kernel(atomic_numbers, pos, edge_index, edge_attr, params):
    raise NotImplementedError("write your pallas kernel here")


# speedup vs baseline: 2.2900x; 2.2900x over previous
import functools

import jax
import jax.numpy as jnp
from jax import lax
from jax.experimental import pallas as pl
from jax.experimental.pallas import tpu as pltpu
from jax.experimental.pallas import tpu_sc as plsc

F32 = jnp.float32
I32 = jnp.int32

NC = 2
NS = 16
NW = NC * NS
L = 16
CH = 80

BN = 2000
BE = 2000


def _silu(v):
    return v * lax.logistic(v)


def _dot(a, b):
    return jnp.dot(a, b, preferred_element_type=F32)


def _tc_embed_body(vocab, a_ref, emb_ref, w1d_ref, w1s_ref, eb1_ref,
                   h_ref, A_ref, B_ref):
    a = a_ref[...]
    iot = lax.broadcasted_iota(I32, (a.shape[0], vocab), 1)
    oh = (a == iot).astype(F32)
    h = _dot(oh, emb_ref[...])
    h_ref[...] = h
    A_ref[...] = _dot(h, w1d_ref[...]) + eb1_ref[...]
    B_ref[...] = _dot(h, w1s_ref[...])


def _tc_embed(a2, emb, w1d, w1s, eb1):
    n, h = a2.shape[0], emb.shape[1]
    vocab = emb.shape[0]
    grid = (n // BN,)
    blk = lambda *shape: shape
    return pl.pallas_call(
        functools.partial(_tc_embed_body, vocab),
        grid=grid,
        in_specs=[
            pl.BlockSpec((BN, 1), lambda i: (i, 0)),
            pl.BlockSpec((vocab, h), lambda i: (0, 0)),
            pl.BlockSpec((h, h), lambda i: (0, 0)),
            pl.BlockSpec((h, h), lambda i: (0, 0)),
            pl.BlockSpec((1, h), lambda i: (0, 0)),
        ],
        out_specs=[
            pl.BlockSpec((BN, h), lambda i: (i, 0)),
            pl.BlockSpec((BN, h), lambda i: (i, 0)),
            pl.BlockSpec((BN, h), lambda i: (i, 0)),
        ],
        out_shape=[
            jax.ShapeDtypeStruct((n, h), F32),
            jax.ShapeDtypeStruct((n, h), F32),
            jax.ShapeDtypeStruct((n, h), F32),
        ],
    )(a2, emb, w1d, w1s, eb1)


def _tc_edge_body(g_ref, xd_ref, ea_ref, wr_ref, we_ref, w2_ref, b2_ref,
                  xw1_ref, xb1_ref, xw2_ref, xb2_ref, infw_ref, infb_ref,
                  om_ref, ox_ref):
    xd = xd_ref[...]
    li = lax.broadcasted_iota(I32, xd.shape, 1)
    r2 = jnp.sum(jnp.where(li < 3, xd * xd, 0.0), axis=1, keepdims=True)
    r = jnp.sqrt(r2)
    ea = ea_ref[...]
    pre = g_ref[...] + r * wr_ref[...] + ea * we_ref[...]
    m = _silu(pre)
    mij = _silu(_dot(m, w2_ref[...]) + b2_ref[...])
    t = _silu(_dot(mij, xw1_ref[...]) + xb1_ref[...])
    wij = jnp.sum(t * xw2_ref[...], axis=1, keepdims=True) + xb2_ref[...]
    infv = lax.logistic(jnp.sum(mij * infw_ref[...], axis=1, keepdims=True)
                        + infb_ref[...])
    om_ref[...] = mij * infv
    ox_ref[...] = wij * xd


def _tc_edge(G, XD, ea2, wr, we, w2, b2, xw1, xb1, xw2t, xb2, infwt, infb):
    e, h = G.shape
    grid = (e // BE,)
    wspec = lambda s0, s1: pl.BlockSpec((s0, s1), lambda i: (0, 0))
    return pl.pallas_call(
        _tc_edge_body,
        grid=grid,
        in_specs=[
            pl.BlockSpec((BE, h), lambda i: (i, 0)),
            pl.BlockSpec((BE, 16), lambda i: (i, 0)),
            pl.BlockSpec((BE, 1), lambda i: (i, 0)),
            wspec(1, h), wspec(1, h), wspec(h, h), wspec(1, h),
            wspec(h, h), wspec(1, h), wspec(1, h), wspec(1, 1),
            wspec(1, h), wspec(1, 1),
        ],
        out_specs=[
            pl.BlockSpec((BE, h), lambda i: (i, 0)),
            pl.BlockSpec((BE, 16), lambda i: (i, 0)),
        ],
        out_shape=[
            jax.ShapeDtypeStruct((e, h), F32),
            jax.ShapeDtypeStruct((e, 16), F32),
        ],
    )(G, XD, ea2, wr, we, w2, b2, xw1, xb1, xw2t, xb2, infwt, infb)


def _tc_node_body(n_blocks, h_ref, xp_ref, pm0_ref, pm1_ref, px0_ref, px1_ref,
                  wh_ref, wm_ref, hb1_ref, hw2_ref, hb2_ref,
                  w1d_ref, w1s_ref, eb1_ref,
                  h_out, xp_out, A_out, B_out):
    h = h_ref[...]
    mi = pm0_ref[...] + pm1_ref[...]
    px = px0_ref[...] + px1_ref[...]
    li = lax.broadcasted_iota(I32, px.shape, 1)
    xagg = jnp.where(li < 3, px, 0.0)
    t2 = _silu(_dot(h, wh_ref[...]) + _dot(mi, wm_ref[...]) + hb1_ref[...])
    hn = h + _dot(t2, hw2_ref[...]) + hb2_ref[...]
    h_out[...] = hn
    xp_out[...] = xp_ref[...] + xagg
    A_out[...] = _dot(hn, w1d_ref[...]) + eb1_ref[...]
    B_out[...] = _dot(hn, w1s_ref[...])


def _tc_node(h, xp, PM, PX, wh, wm, hb1, hw2, hb2, w1d, w1s, eb1):
    n, hd = h.shape
    nb = n // BN
    grid = (nb,)
    wspec = lambda s0, s1: pl.BlockSpec((s0, s1), lambda i: (0, 0))
    return pl.pallas_call(
        functools.partial(_tc_node_body, nb),
        grid=grid,
        in_specs=[
            pl.BlockSpec((BN, hd), lambda i: (i, 0)),
            pl.BlockSpec((BN, 16), lambda i: (i, 0)),
            pl.BlockSpec((BN, hd), lambda i: (i, 0)),
            pl.BlockSpec((BN, hd), lambda i, _nb=nb: (i + _nb, 0)),
            pl.BlockSpec((BN, 16), lambda i: (i, 0)),
            pl.BlockSpec((BN, 16), lambda i, _nb=nb: (i + _nb, 0)),
            wspec(hd, hd), wspec(hd, hd), wspec(1, hd), wspec(hd, hd),
            wspec(1, hd), wspec(hd, hd), wspec(hd, hd), wspec(1, hd),
        ],
        out_specs=[
            pl.BlockSpec((BN, hd), lambda i: (i, 0)),
            pl.BlockSpec((BN, 16), lambda i: (i, 0)),
            pl.BlockSpec((BN, hd), lambda i: (i, 0)),
            pl.BlockSpec((BN, hd), lambda i: (i, 0)),
        ],
        out_shape=[
            jax.ShapeDtypeStruct((n, hd), F32),
            jax.ShapeDtypeStruct((n, 16), F32),
            jax.ShapeDtypeStruct((n, hd), F32),
            jax.ShapeDtypeStruct((n, hd), F32),
        ],
    )(h, xp, PM, PM, PX, PX, wh, wm, hb1, hw2, hb2, w1d, w1s, eb1)


def _sc_gather_body(e, h, a_hbm, b_hbm, xq_hbm, src_hbm, dst_hbm,
                    g_hbm, xd_hbm,
                    srcv, dstv, qsv, qdv, ra, rb, xs, xdv, xdd,
                    sem1, sem2, sem3, sem4):
    c = lax.axis_index("c")
    s = lax.axis_index("s")
    wid = s * NC + c
    ew = e // NW
    nch = ew // CH
    base0 = wid * ew

    def chunk(ci, carry):
        base = base0 + ci * CH
        pltpu.sync_copy(src_hbm.at[pl.ds(base, CH)], srcv)
        pltpu.sync_copy(dst_hbm.at[pl.ds(base, CH)], dstv)
        for g in range(CH // L):
            sl = pl.ds(g * L, L)
            qsv[sl] = lax.shift_right_logical(srcv[sl], 3)
            qdv[sl] = lax.shift_right_logical(dstv[sl], 3)
        cp1 = pltpu.async_copy(a_hbm.at[dstv], ra, sem1)
        cp2 = pltpu.async_copy(b_hbm.at[srcv], rb, sem2)
        cp3 = pltpu.async_copy(xq_hbm.at[qsv], xs, sem3)
        cp4 = pltpu.async_copy(xq_hbm.at[qdv], xdv, sem4)
        cp1.wait()
        cp2.wait()
        cp3.wait()
        cp4.wait()

        lane = lax.iota(I32, L)

        def row(i, rc):
            for j in range(h // L):
                sl = pl.ds(j * L, L)
                ra[i, sl] = ra[i, sl] + rb[i, sl]
            ilo = i & ~(L - 1)
            sel = lane == (i - ilo)
            sv = jnp.sum(jnp.where(sel, srcv[pl.ds(ilo, L)], 0))
            dv = jnp.sum(jnp.where(sel, dstv[pl.ds(ilo, L)], 0))
            os = (sv & 7) * L
            od = (dv & 7) * L
            xdd[i, :] = xs[i, pl.ds(os, L)] - xdv[i, pl.ds(od, L)]
            return rc

        lax.fori_loop(0, CH, row, 0)
        pltpu.sync_copy(ra, g_hbm.at[pl.ds(base, CH)])
        pltpu.sync_copy(xdd, xd_hbm.at[pl.ds(base, CH)])
        return carry

    lax.fori_loop(0, nch, chunk, 0)


def _sc_gather(A, B, xq, src, dst):
    n, h = A.shape
    e = src.shape[0]
    mesh = plsc.VectorSubcoreMesh(core_axis_name="c", subcore_axis_name="s")
    return pl.kernel(
        functools.partial(_sc_gather_body, e, h),
        out_type=(
            jax.ShapeDtypeStruct((e, h), F32),
            jax.ShapeDtypeStruct((e, 16), F32),
        ),
        mesh=mesh,
        scratch_types=[
            pltpu.VMEM((CH,), I32),
            pltpu.VMEM((CH,), I32),
            pltpu.VMEM((CH,), I32),
            pltpu.VMEM((CH,), I32),
            pltpu.VMEM((CH, h), F32),
            pltpu.VMEM((CH, h), F32),
            pltpu.VMEM((CH, h), F32),
            pltpu.VMEM((CH, h), F32),
            pltpu.VMEM((CH, 16), F32),
            pltpu.SemaphoreType.DMA,
            pltpu.SemaphoreType.DMA,
            pltpu.SemaphoreType.DMA,
            pltpu.SemaphoreType.DMA,
        ],
        compiler_params=pltpu.CompilerParams(needs_layout_passes=False),
    )(A, B, xq, src, dst)


def _sc_scatter_body(e, n, h, om_hbm, ox_hbm, dst_hbm, pm_hbm, px_hbm,
                     accm, accx, dstv, idxv, bm, bx, semg):
    c = lax.axis_index("c")
    s = lax.axis_index("s")
    wid = s * NC + c
    ew = e // NW
    nch = ew // CH
    base0 = wid * ew
    nrc = n // CH
    nz = (nrc - s + NS - 1) // NS
    lane = lax.iota(I32, L)

    def zrow(i, carry):
        for j in range(h // L):
            bm[i, pl.ds(j * L, L)] = jnp.zeros((L,), F32)
        bx[i, :] = jnp.zeros((L,), F32)
        return carry

    lax.fori_loop(0, CH, zrow, 0)

    def fill_idx(base):
        for g in range(CH // L):
            idxv[pl.ds(g * L, L)] = lane + (base + g * L)

    def zblk(k, carry):
        fill_idx((s + k * NS) * CH)
        pltpu.sync_copy(bm, accm.at[idxv])
        pltpu.sync_copy(bx, accx.at[idxv])
        return carry

    lax.fori_loop(0, nz, zblk, 0)
    plsc.subcore_barrier()

    def chunk(ci, carry):
        base = base0 + ci * CH
        pltpu.sync_copy(dst_hbm.at[pl.ds(base, CH)], dstv)
        pltpu.sync_copy(om_hbm.at[pl.ds(base, CH)], bm)
        pltpu.sync_copy(ox_hbm.at[pl.ds(base, CH)], bx)
        pltpu.sync_copy(bm, accm.at[dstv], add=True)
        pltpu.sync_copy(bx, accx.at[dstv], add=True)
        return carry

    lax.fori_loop(0, nch, chunk, 0)
    plsc.subcore_barrier()

    def wblk(k, carry):
        base = (s + k * NS) * CH
        fill_idx(base)
        pltpu.async_copy(accm.at[idxv], bm, semg).wait()
        pltpu.sync_copy(bm, pm_hbm.at[pl.ds(c * n + base, CH)])
        pltpu.async_copy(accx.at[idxv], bx, semg).wait()
        pltpu.sync_copy(bx, px_hbm.at[pl.ds(c * n + base, CH)])
        return carry

    lax.fori_loop(0, nz, wblk, 0)


def _sc_scatter(OM, OX, dst, n):
    e, h = OM.shape
    mesh = plsc.VectorSubcoreMesh(core_axis_name="c", subcore_axis_name="s")
    return pl.kernel(
        functools.partial(_sc_scatter_body, e, n, h),
        out_type=(
            jax.ShapeDtypeStruct((NC * n, h), F32),
            jax.ShapeDtypeStruct((NC * n, 16), F32),
        ),
        mesh=mesh,
        scratch_types=[
            pltpu.VMEM_SHARED((n, h), F32),
            pltpu.VMEM_SHARED((n, 16), F32),
            pltpu.VMEM((CH,), I32),
            pltpu.VMEM((CH,), I32),
            pltpu.VMEM((CH, h), F32),
            pltpu.VMEM((CH, 16), F32),
            pltpu.SemaphoreType.DMA,
        ],
        compiler_params=pltpu.CompilerParams(needs_layout_passes=False),
    )(OM, OX, dst)


def kernel(atomic_numbers, pos, edge_index, edge_attr, params):
    emb = params["emb"]
    layers = params["layers"]
    n = pos.shape[0]
    hd = emb.shape[1]
    e = edge_index.shape[1]

    src = edge_index[0]
    dst = edge_index[1]
    xpad = jnp.pad(pos, ((0, 0), (0, 13)))
    a2 = atomic_numbers.reshape(n, 1)

    def wprep(lp):
        w1 = lp["e_W1"]
        return dict(
            w1d=w1[:hd], w1s=w1[hd:2 * hd],
            wr=w1[2 * hd:2 * hd + 1], we=w1[2 * hd + 1:2 * hd + 2],
            eb1=lp["e_b1"].reshape(1, hd),
            w2=lp["e_W2"], b2=lp["e_b2"].reshape(1, hd),
            xw1=lp["x_W1"], xb1=lp["x_b1"].reshape(1, hd),
            xw2t=lp["x_W2"].reshape(1, hd), xb2=lp["x_b2"].reshape(1, 1),
            infwt=lp["inf_W"].reshape(1, hd), infb=lp["inf_b"].reshape(1, 1),
            wh=lp["h_W1"][:hd], wm=lp["h_W1"][hd:],
            hb1=lp["h_b1"].reshape(1, hd),
            hw2=lp["h_W2"], hb2=lp["h_b2"].reshape(1, hd),
        )

    wl = [wprep(lp) for lp in layers]

    h, A, B = _tc_embed(a2, emb, wl[0]["w1d"], wl[0]["w1s"], wl[0]["eb1"])
    nl = len(layers)
    for l in range(nl):
        w = wl[l]
        wn = wl[(l + 1) % nl]
        G, XD = _sc_gather(A, B, xpad.reshape(n // 8, 8 * 16), src, dst)
        OM, OX = _tc_edge(G, XD, edge_attr, w["wr"], w["we"], w["w2"], w["b2"],
                          w["xw1"], w["xb1"], w["xw2t"], w["xb2"],
                          w["infwt"], w["infb"])
        PM, PX = _sc_scatter(OM, OX, dst, n)
        h, xpad, A, B = _tc_node(h, xpad, PM, PX,
                                 w["wh"], w["wm"], w["hb1"], w["hw2"], w["hb2"],
                                 wn["w1d"], wn["w1s"], wn["eb1"])
    return h, xpad[:, :3]
